# speedup vs baseline: 6.8785x; 6.8785x over previous
import functools

import jax
import jax.numpy as jnp
from jax import lax
from jax.experimental import pallas as pl
from jax.experimental.pallas import tpu as pltpu
from jax.experimental.pallas import tpu_sc as plsc

NUM_ROWS = 100000
D = 128
B = 1024
S = 50
T = 5

NC = 2
NS = 16
NW = NC * NS
R = B * S
R_W = R // NW
G = 80
NG = R_W // G
LANES = D // 16


def _sc_body(idx_hbm, table_hbm, out_hbm, idx_v, stg_v, outb_v, gsem, wsem):
    wid = lax.axis_index("s") * NC + lax.axis_index("c")
    base = wid * R_W

    pltpu.sync_copy(idx_hbm.at[:, wid], idx_v)

    for g in range(NG):
        for j in range(T):
            pltpu.async_copy(table_hbm.at[idx_v.at[j, g]], stg_v.at[j], gsem)
        for j in range(T):
            pltpu.make_async_copy(table_hbm.at[idx_v.at[j, g]], stg_v.at[j],
                                  gsem).wait()

        if g > 0:
            pltpu.make_async_copy(outb_v, out_hbm.at[pl.ds(base, G)],
                                  wsem).wait()

        def row_sum(r, carry):
            for c in range(LANES):
                col = pl.ds(c * 16, 16)
                acc = stg_v[0, r, col]
                for j in range(1, T):
                    acc = acc + stg_v[j, r, col]
                outb_v[r, col] = acc
            return carry

        lax.fori_loop(0, G, row_sum, 0, unroll=2)

        pltpu.async_copy(outb_v, out_hbm.at[pl.ds(base + g * G, G)], wsem)

    pltpu.make_async_copy(outb_v, out_hbm.at[pl.ds(base, G)], wsem).wait()


def kernel(morphemes, table):
    idx = morphemes.astype(jnp.int32).reshape(R, T).T.reshape(T, NW, NG, G)

    sc_kernel = pl.kernel(
        _sc_body,
        out_type=jax.ShapeDtypeStruct((R, D), jnp.float32),
        mesh=plsc.VectorSubcoreMesh(core_axis_name="c", subcore_axis_name="s"),
        scratch_types=[
            pltpu.VMEM((T, NG, G), jnp.int32),
            pltpu.VMEM((T, G, D), jnp.float32),
            pltpu.VMEM((G, D), jnp.float32),
            pltpu.SemaphoreType.DMA,
            pltpu.SemaphoreType.DMA,
        ],
    )
    out = sc_kernel(idx, table)
    return out.reshape(B, S, D)


# speedup vs baseline: 8.8308x; 1.2838x over previous
import functools

import jax
import jax.numpy as jnp
from jax import lax
from jax.experimental import pallas as pl
from jax.experimental.pallas import tpu as pltpu
from jax.experimental.pallas import tpu_sc as plsc

NUM_ROWS = 100000
D = 128
B = 1024
S = 50
T = 5

NC = 2
NS = 16
NW = NC * NS
R = B * S
R_W = R // NW
G = 64
NG = R_W // G
LANES = D // 16


def _sc_body(idx_hbm, table_hbm, out_hbm, idx_v, stg_v, outb_v,
             gsem0, gsem1, wsem0, wsem1):
    wid = lax.axis_index("s") * NC + lax.axis_index("c")
    base = wid * R_W
    gsems = (gsem0, gsem1)
    wsems = (wsem0, wsem1)

    pltpu.sync_copy(idx_hbm.at[:, wid], idx_v)

    def fire(g, buf):
        for j in range(T):
            pltpu.async_copy(table_hbm.at[idx_v.at[j, g]], stg_v.at[buf, j],
                             gsems[buf])

    def drain(g, buf):
        for j in range(T):
            pltpu.make_async_copy(table_hbm.at[idx_v.at[j, g]],
                                  stg_v.at[buf, j], gsems[buf]).wait()

    fire(0, 0)
    for g in range(NG):
        buf = g % 2
        if g + 1 < NG:
            fire(g + 1, 1 - buf)
        drain(g, buf)

        if g >= 2:
            pltpu.make_async_copy(outb_v.at[buf], out_hbm.at[pl.ds(base, G)],
                                  wsems[buf]).wait()

        def row_sum(r, carry):
            for c in range(LANES):
                col = pl.ds(c * 16, 16)
                acc = stg_v[buf, 0, r, col]
                for j in range(1, T):
                    acc = acc + stg_v[buf, j, r, col]
                outb_v[buf, r, col] = acc
            return carry

        lax.fori_loop(0, G, row_sum, 0, unroll=2)

        pltpu.async_copy(outb_v.at[buf], out_hbm.at[pl.ds(base + g * G, G)],
                         wsems[buf])

    for buf in range(2):
        pltpu.make_async_copy(outb_v.at[buf], out_hbm.at[pl.ds(base, G)],
                              wsems[buf]).wait()


def kernel(morphemes, table):
    idx = morphemes.astype(jnp.int32).reshape(R, T).T.reshape(T, NW, NG, G)

    sc_kernel = pl.kernel(
        _sc_body,
        out_type=jax.ShapeDtypeStruct((R, D), jnp.float32),
        mesh=plsc.VectorSubcoreMesh(core_axis_name="c", subcore_axis_name="s"),
        scratch_types=[
            pltpu.VMEM((T, NG, G), jnp.int32),
            pltpu.VMEM((2, T, G, D), jnp.float32),
            pltpu.VMEM((2, G, D), jnp.float32),
            pltpu.SemaphoreType.DMA,
            pltpu.SemaphoreType.DMA,
            pltpu.SemaphoreType.DMA,
            pltpu.SemaphoreType.DMA,
        ],
    )
    out = sc_kernel(idx, table)
    return out.reshape(B, S, D)
